# baseline (speedup 1.0000x reference)
import functools

import jax
import jax.numpy as jnp
from jax.experimental import pallas as pl
from jax.experimental.pallas import tpu as pltpu
from jax.experimental.pallas import tpu_sc as plsc

COMMITMENT_COST = 0.25
EPSILON = 1e-05


def _vq_main(z_ref, emb_ref, e2_ref, ph_ref, cnt_ref, idx_ref, *, T, M):
    r = pl.program_id(1)
    x = z_ref[0]
    emb = emb_ref[...]
    RB = x.shape[0]

    x2 = jnp.sum(x * x, axis=1, keepdims=True)
    xz = jax.lax.dot_general(x, emb, (((1,), (1,)), ((), ())),
                             preferred_element_type=jnp.float32)
    dist = e2_ref[...] + x2 - 2.0 * xz

    dmin = jnp.min(dist, axis=1, keepdims=True)
    ee = jnp.exp(jnp.sqrt(dmin) - jnp.sqrt(dist))
    s = jnp.sum(ee, axis=1, keepdims=True)
    recip = jnp.transpose((1.0 / T) / s)
    php = jax.lax.dot_general(recip, ee, (((1,), (0,)), ((), ())),
                              preferred_element_type=jnp.float32)

    iota = jax.lax.broadcasted_iota(jnp.int32, (RB, M), 1)
    idxm = jnp.where(dist == dmin, iota, M)
    idx = jnp.min(idxm, axis=1, keepdims=True)
    onehot = (iota == idx).astype(jnp.float32)
    ones_row = jnp.ones((1, RB), jnp.float32)
    cnt = jax.lax.dot_general(ones_row, onehot, (((1,), (0,)), ((), ())),
                              preferred_element_type=jnp.float32)

    idx_ref[0] = idx

    @pl.when(r == 0)
    def _():
        ph_ref[...] = jnp.zeros_like(ph_ref)
        cnt_ref[...] = jnp.zeros_like(cnt_ref)

    ph_ref[0] += php
    cnt_ref[0] += cnt


def _sc_gather(embedding, idx_flat, n, d):
    W = 128
    mesh = plsc.VectorSubcoreMesh(core_axis_name="c", subcore_axis_name="s")

    @functools.partial(
        pl.kernel,
        out_type=jax.ShapeDtypeStruct((n, d), jnp.float32),
        mesh=mesh,
    )
    def _gather_kernel(emb_hbm, i_hbm, o_hbm):
        def body(i_vmem, o_vmem):
            pltpu.sync_copy(emb_hbm.at[i_vmem.at[0]], o_vmem)

        pltpu.emit_pipeline(
            body,
            grid=(n // W,),
            in_specs=[pl.BlockSpec((1, W), lambda i: (0, i))],
            out_specs=[pl.BlockSpec((W, d), lambda i: (i, 0))],
            core_axis_name=("c", "s"),
            dimension_semantics=(pltpu.PARALLEL,),
        )(i_hbm, o_hbm)

    return _gather_kernel(embedding, idx_flat.reshape(1, n))


def kernel(audio_semantic, eeg_semantic, embedding):
    B, T, D = audio_semantic.shape
    M = embedding.shape[0]
    BT = B * T
    RB = 288
    nr = T // RB
    G = 2 * B

    z = jnp.concatenate([audio_semantic, eeg_semantic], axis=0)
    e2 = jnp.sum(embedding * embedding, axis=1)[None, :]

    ph, cnt, idx = pl.pallas_call(
        functools.partial(_vq_main, T=T, M=M),
        grid=(G, nr),
        in_specs=[
            pl.BlockSpec((1, RB, D), lambda g, r: (g, r, 0)),
            pl.BlockSpec((M, D), lambda g, r: (0, 0)),
            pl.BlockSpec((1, M), lambda g, r: (0, 0)),
        ],
        out_specs=[
            pl.BlockSpec((1, 1, M), lambda g, r: (g, 0, 0)),
            pl.BlockSpec((1, 1, M), lambda g, r: (g, 0, 0)),
            pl.BlockSpec((1, RB, 1), lambda g, r: (g, r, 0)),
        ],
        out_shape=[
            jax.ShapeDtypeStruct((G, 1, M), jnp.float32),
            jax.ShapeDtypeStruct((G, 1, M), jnp.float32),
            jax.ShapeDtypeStruct((G, T, 1), jnp.int32),
        ],
    )(z, embedding, e2)

    a_ph, e_ph = ph[:B, 0], ph[B:, 0]
    a_counts, e_counts = cnt[:B, 0], cnt[B:, 0]
    q = _sc_gather(embedding, idx, G * T, D).reshape(G, T, D)
    a_q, e_q = q[:B], q[B:]

    Scode = a_ph @ jnp.log(e_ph.T + 1e-10) + e_ph @ jnp.log(a_ph.T + 1e-10)
    MaxScode = jnp.max(-Scode)
    EScode = jnp.exp(Scode + MaxScode)
    EScode_dim1sum = jnp.sum(EScode, axis=1)
    Lcmcm = -jnp.sum(jnp.log(jnp.diagonal(EScode) / (EScode_dim1sum + EPSILON))) / B
    cmcm_loss = 0.5 * Lcmcm

    a_mode = jnp.argmax(a_counts, axis=1)
    e_mode = jnp.argmax(e_counts, axis=1)
    equal_num = jnp.sum(a_mode == e_mode)

    def mse(x, y):
        return jnp.mean((x - y) ** 2)

    a_e_latent_loss = mse(audio_semantic, a_q)
    ae_e_latent_loss = mse(audio_semantic, e_q)
    a_loss = COMMITMENT_COST * (2.0 * a_e_latent_loss + ae_e_latent_loss)
    e_e_latent_loss = mse(eeg_semantic, e_q)
    ea_e_latent_loss = mse(eeg_semantic, a_q)
    e_loss = COMMITMENT_COST * (2.0 * e_e_latent_loss + ea_e_latent_loss)

    a_quantized_st = audio_semantic + (a_q - audio_semantic)
    e_quantized_st = eeg_semantic + (e_q - eeg_semantic)

    a_avg_probs = jnp.sum(a_counts, axis=0) / BT
    a_perplexity = jnp.exp(-jnp.sum(a_avg_probs * jnp.log(a_avg_probs + 1e-10)))
    e_avg_probs = jnp.sum(e_counts, axis=0) / BT
    e_perplexity = jnp.exp(-jnp.sum(e_avg_probs * jnp.log(e_avg_probs + 1e-10)))

    return (a_quantized_st, e_quantized_st, a_loss, e_loss,
            a_perplexity, e_perplexity, cmcm_loss, equal_num)


# speedup vs baseline: 1.0142x; 1.0142x over previous
import functools

import jax
import jax.numpy as jnp
from jax.experimental import pallas as pl
from jax.experimental.pallas import tpu as pltpu
from jax.experimental.pallas import tpu_sc as plsc

COMMITMENT_COST = 0.25
EPSILON = 1e-05


def _vq_main(z_ref, emb_ref, e2_ref, ph_ref, cnt_ref, idx_ref, *, T, M):
    r = pl.program_id(1)
    x = z_ref[0]
    emb = emb_ref[...]
    RB = x.shape[0]

    x2 = jnp.sum(x * x, axis=1, keepdims=True)
    xz = jax.lax.dot_general(x, emb, (((1,), (1,)), ((), ())),
                             preferred_element_type=jnp.float32)
    dist = e2_ref[...] + x2 - 2.0 * xz

    dmin = jnp.min(dist, axis=1, keepdims=True)
    ee = jnp.exp(jnp.sqrt(dmin) - jnp.sqrt(dist))
    s = jnp.sum(ee, axis=1, keepdims=True)
    recip = jnp.transpose((1.0 / T) / s)
    php = jax.lax.dot_general(recip, ee, (((1,), (0,)), ((), ())),
                              preferred_element_type=jnp.float32)

    iota = jax.lax.broadcasted_iota(jnp.int32, (RB, M), 1)
    idxm = jnp.where(dist == dmin, iota, M)
    idx = jnp.min(idxm, axis=1, keepdims=True)
    onehot = (iota == idx).astype(jnp.float32)
    ones_row = jnp.ones((1, RB), jnp.float32)
    cnt = jax.lax.dot_general(ones_row, onehot, (((1,), (0,)), ((), ())),
                              preferred_element_type=jnp.float32)

    idx_ref[0] = idx

    @pl.when(r == 0)
    def _():
        ph_ref[...] = jnp.zeros_like(ph_ref)
        cnt_ref[...] = jnp.zeros_like(cnt_ref)

    ph_ref[0] += php
    cnt_ref[0] += cnt


def _sc_gather(embedding, idx_flat, n, d):
    W = 128
    mesh = plsc.VectorSubcoreMesh(core_axis_name="c", subcore_axis_name="s")

    @functools.partial(
        pl.kernel,
        out_type=jax.ShapeDtypeStruct((n, d), jnp.float32),
        mesh=mesh,
    )
    def _gather_kernel(emb_hbm, i_hbm, o_hbm):
        def body(i_vmem, o_vmem):
            pltpu.sync_copy(emb_hbm.at[i_vmem.at[0]], o_vmem)

        pltpu.emit_pipeline(
            body,
            grid=(n // W,),
            in_specs=[pl.BlockSpec((1, W), lambda i: (0, i))],
            out_specs=[pl.BlockSpec((W, d), lambda i: (i, 0))],
            core_axis_name=("c", "s"),
            dimension_semantics=(pltpu.PARALLEL,),
        )(i_hbm, o_hbm)

    return _gather_kernel(embedding, idx_flat.reshape(1, n))


def _epilogue(z_ref, q_ref, ph_ref, cnt_ref,
              sta_ref, ste_ref, aloss_ref, eloss_ref, aperp_ref, eperp_ref,
              cmcm_ref, eq_ref, *, B, T, D, M):
    z = z_ref[...]
    q = q_ref[...]

    n = B * T * D
    a, e = z[:B], z[B:]
    aq, eq = q[:B], q[B:]
    sta_ref[...] = a + (aq - a)
    ste_ref[...] = e + (eq - e)

    def mse(x, y):
        d = x - y
        return jnp.sum(d * d) * (1.0 / n)

    aloss_ref[...] = jnp.reshape(COMMITMENT_COST * (2.0 * mse(a, aq) + mse(a, eq)), (1, 1))
    eloss_ref[...] = jnp.reshape(COMMITMENT_COST * (2.0 * mse(e, eq) + mse(e, aq)), (1, 1))

    a_ph = ph_ref[:B, 0, :]
    e_ph = ph_ref[B:, 0, :]
    la = jnp.log(a_ph + 1e-10)
    le = jnp.log(e_ph + 1e-10)
    Scode = (jax.lax.dot_general(a_ph, le, (((1,), (1,)), ((), ())),
                                 preferred_element_type=jnp.float32)
             + jax.lax.dot_general(e_ph, la, (((1,), (1,)), ((), ())),
                                   preferred_element_type=jnp.float32))
    MaxScode = jnp.max(-Scode)
    EScode = jnp.exp(Scode + MaxScode)
    rsum = jnp.sum(EScode, axis=1, keepdims=True)
    ri = jax.lax.broadcasted_iota(jnp.int32, (B, B), 0)
    ci = jax.lax.broadcasted_iota(jnp.int32, (B, B), 1)
    diag = jnp.sum(jnp.where(ri == ci, EScode, 0.0), axis=1, keepdims=True)
    Lcmcm = -jnp.sum(jnp.log(diag / (rsum + EPSILON))) * (1.0 / B)
    cmcm_ref[...] = jnp.reshape(0.5 * Lcmcm, (1, 1))

    a_cnt = cnt_ref[:B, 0, :]
    e_cnt = cnt_ref[B:, 0, :]
    iota = jax.lax.broadcasted_iota(jnp.int32, (B, M), 1)
    amax = jnp.max(a_cnt, axis=1, keepdims=True)
    emax = jnp.max(e_cnt, axis=1, keepdims=True)
    a_mode = jnp.min(jnp.where(a_cnt == amax, iota, M), axis=1, keepdims=True)
    e_mode = jnp.min(jnp.where(e_cnt == emax, iota, M), axis=1, keepdims=True)
    eq_ref[...] = jnp.reshape(jnp.sum((a_mode == e_mode).astype(jnp.int32)), (1, 1))

    inv_n = 1.0 / (B * T)
    pa = jnp.sum(a_cnt, axis=0, keepdims=True) * inv_n
    pe = jnp.sum(e_cnt, axis=0, keepdims=True) * inv_n
    aperp_ref[...] = jnp.reshape(jnp.exp(-jnp.sum(pa * jnp.log(pa + 1e-10))), (1, 1))
    eperp_ref[...] = jnp.reshape(jnp.exp(-jnp.sum(pe * jnp.log(pe + 1e-10))), (1, 1))


def kernel(audio_semantic, eeg_semantic, embedding):
    B, T, D = audio_semantic.shape
    M = embedding.shape[0]
    BT = B * T
    RB = 288
    nr = T // RB
    G = 2 * B

    z = jnp.concatenate([audio_semantic, eeg_semantic], axis=0)
    e2 = jnp.sum(embedding * embedding, axis=1)[None, :]

    ph, cnt, idx = pl.pallas_call(
        functools.partial(_vq_main, T=T, M=M),
        grid=(G, nr),
        in_specs=[
            pl.BlockSpec((1, RB, D), lambda g, r: (g, r, 0)),
            pl.BlockSpec((M, D), lambda g, r: (0, 0)),
            pl.BlockSpec((1, M), lambda g, r: (0, 0)),
        ],
        out_specs=[
            pl.BlockSpec((1, 1, M), lambda g, r: (g, 0, 0)),
            pl.BlockSpec((1, 1, M), lambda g, r: (g, 0, 0)),
            pl.BlockSpec((1, RB, 1), lambda g, r: (g, r, 0)),
        ],
        out_shape=[
            jax.ShapeDtypeStruct((G, 1, M), jnp.float32),
            jax.ShapeDtypeStruct((G, 1, M), jnp.float32),
            jax.ShapeDtypeStruct((G, T, 1), jnp.int32),
        ],
    )(z, embedding, e2)

    q = _sc_gather(embedding, idx, G * T, D).reshape(G, T, D)

    sta, ste, a_loss, e_loss, a_perp, e_perp, cmcm, eqn = pl.pallas_call(
        functools.partial(_epilogue, B=B, T=T, D=D, M=M),
        grid=(1,),
        in_specs=[
            pl.BlockSpec((G, T, D), lambda i: (0, 0, 0)),
            pl.BlockSpec((G, T, D), lambda i: (0, 0, 0)),
            pl.BlockSpec((G, 1, M), lambda i: (0, 0, 0)),
            pl.BlockSpec((G, 1, M), lambda i: (0, 0, 0)),
        ],
        out_specs=[
            pl.BlockSpec((B, T, D), lambda i: (0, 0, 0)),
            pl.BlockSpec((B, T, D), lambda i: (0, 0, 0)),
            pl.BlockSpec((1, 1), lambda i: (0, 0)),
            pl.BlockSpec((1, 1), lambda i: (0, 0)),
            pl.BlockSpec((1, 1), lambda i: (0, 0)),
            pl.BlockSpec((1, 1), lambda i: (0, 0)),
            pl.BlockSpec((1, 1), lambda i: (0, 0)),
            pl.BlockSpec((1, 1), lambda i: (0, 0)),
        ],
        out_shape=[
            jax.ShapeDtypeStruct((B, T, D), jnp.float32),
            jax.ShapeDtypeStruct((B, T, D), jnp.float32),
            jax.ShapeDtypeStruct((1, 1), jnp.float32),
            jax.ShapeDtypeStruct((1, 1), jnp.float32),
            jax.ShapeDtypeStruct((1, 1), jnp.float32),
            jax.ShapeDtypeStruct((1, 1), jnp.float32),
            jax.ShapeDtypeStruct((1, 1), jnp.float32),
            jax.ShapeDtypeStruct((1, 1), jnp.int32),
        ],
    )(z, q, ph, cnt)

    return (sta, ste, a_loss[0, 0], e_loss[0, 0],
            a_perp[0, 0], e_perp[0, 0], cmcm[0, 0], eqn[0, 0])
